# baseline (speedup 1.0000x reference)
import functools

import jax
import jax.numpy as jnp
from jax import lax
from jax.experimental import pallas as pl
from jax.experimental.pallas import tpu as pltpu
from jax.experimental.pallas import tpu_sc as plsc

EMB = 512
OUT = 512
NW = 32
CH = 120
M_BLK = 512


def _make_sc_gather(b_total):
  b_per_w = b_total // NW
  nchunk = b_per_w // CH
  assert b_per_w % CH == 0 and b_total % (8 * NW) == 0

  mesh = plsc.VectorSubcoreMesh(core_axis_name="c", subcore_axis_name="s")

  @functools.partial(
      pl.kernel,
      mesh=mesh,
      out_type=[
          jax.ShapeDtypeStruct((b_total, EMB), jnp.float32),
          jax.ShapeDtypeStruct((b_total,), jnp.int32),
      ],
      scratch_types=[
          pltpu.VMEM((b_per_w,), jnp.int32),
          pltpu.VMEM((2, CH, EMB), jnp.float32),
          pltpu.VMEM((b_per_w,), jnp.int32),
          pltpu.SemaphoreType.DMA,
          pltpu.SemaphoreType.DMA,
          pltpu.SemaphoreType.DMA,
          pltpu.SemaphoreType.DMA,
      ],
  )
  def gather_kernel(table_hbm, posmap_hbm, ids_hbm, rows_out_hbm, pos_out_hbm,
                    idx_v, rows_v, pos_v, gsem, psem, osem0, osem1):
    wid = lax.axis_index("s") * 2 + lax.axis_index("c")
    base = wid * b_per_w
    pltpu.sync_copy(ids_hbm.at[pl.ds(base, b_per_w)], idx_v)
    osems = (osem0, osem1)
    pending = [None, None]
    for c in range(nchunk):
      s = c % 2
      if pending[s] is not None:
        pending[s].wait()
      idx_c = idx_v.at[pl.ds(c * CH, CH)]
      gp = pltpu.async_copy(posmap_hbm.at[idx_c],
                            pos_v.at[pl.ds(c * CH, CH)], psem)
      g = pltpu.async_copy(table_hbm.at[idx_c], rows_v.at[s], gsem)
      gp.wait()
      g.wait()
      pending[s] = pltpu.async_copy(
          rows_v.at[s], rows_out_hbm.at[pl.ds(base + c * CH, CH)], osems[s])
    for p in pending:
      if p is not None:
        p.wait()
    pltpu.sync_copy(pos_v, pos_out_hbm.at[pl.ds(base, b_per_w)])

  return gather_kernel


def _project_block(x_ref, pidx_ref, poshead_ref, weh_ref, wel_ref, wp_ref,
                   b_ref, o_ref):
  x = x_ref[...]
  xh = x.astype(jnp.bfloat16)
  xl = (x - xh.astype(jnp.float32)).astype(jnp.bfloat16)
  wh = weh_ref[...]
  wl = wel_ref[...]
  dims = (((1,), (1,)), ((), ()))
  acc = lax.dot_general(xh, wh, dims, preferred_element_type=jnp.float32)
  acc = acc + lax.dot_general(xh, wl, dims,
                              preferred_element_type=jnp.float32)
  acc = acc + lax.dot_general(xl, wh, dims,
                              preferred_element_type=jnp.float32)
  pidx = pidx_ref[0, 0, :]
  ph = (pidx[:, None] == lax.broadcasted_iota(jnp.int32, (M_BLK, 16), 1)
        ).astype(jnp.float32)
  pp = lax.dot_general(poshead_ref[...], wp_ref[...], dims,
                       preferred_element_type=jnp.float32)
  acc = acc + lax.dot_general(ph, pp, (((1,), (0,)), ((), ())),
                              preferred_element_type=jnp.float32)
  o_ref[...] = acc + b_ref[...]


def _project(x, pidx3, poshead, weh, wel, wp, b2):
  n = x.shape[0]
  nb = n // M_BLK
  pdim = wp.shape[1]
  return pl.pallas_call(
      _project_block,
      grid=(nb,),
      in_specs=[
          pl.BlockSpec((M_BLK, EMB), lambda i: (i, 0)),
          pl.BlockSpec((1, 1, M_BLK), lambda i: (i, 0, 0)),
          pl.BlockSpec((16, pdim), lambda i: (0, 0)),
          pl.BlockSpec((OUT, EMB), lambda i: (0, 0)),
          pl.BlockSpec((OUT, EMB), lambda i: (0, 0)),
          pl.BlockSpec((OUT, pdim), lambda i: (0, 0)),
          pl.BlockSpec((1, OUT), lambda i: (0, 0)),
      ],
      out_specs=pl.BlockSpec((M_BLK, OUT), lambda i: (i, 0)),
      out_shape=jax.ShapeDtypeStruct((n, OUT), jnp.float32),
      compiler_params=pltpu.CompilerParams(
          dimension_semantics=("arbitrary",)),
  )(x, pidx3, poshead, weh, wel, wp, b2)


def kernel(entity_ids, entity_table, pos_table, entity_id_to_pos_index, W, b):
  nb, nc, ne = entity_ids.shape
  n = entity_ids.size
  ids = jnp.transpose(entity_ids, (0, 2, 1)).reshape(-1).astype(jnp.int32)
  posmap = entity_id_to_pos_index.astype(jnp.int32)
  rows, pidx = _make_sc_gather(n)(entity_table, posmap, ids)
  pidx3 = pidx.reshape(n // M_BLK, 1, M_BLK)
  we = W[:, :EMB]
  weh = we.astype(jnp.bfloat16)
  wel = (we - weh.astype(jnp.float32)).astype(jnp.bfloat16)
  wp = W[:, EMB:]
  poshead = pos_table[:16]
  out = _project(rows, pidx3, poshead, weh, wel, wp, b.reshape(1, OUT))
  return out.reshape(nb, ne, nc, OUT).transpose(0, 2, 1, 3)


# speedup vs baseline: 1.2541x; 1.2541x over previous
import functools

import jax
import jax.numpy as jnp
from jax import lax
from jax.experimental import pallas as pl
from jax.experimental.pallas import tpu as pltpu
from jax.experimental.pallas import tpu_sc as plsc

EMB = 512
OUT = 512
NW = 32
CH = 120
M_BLK = 512
K_SPLIT = 4


def _make_sc_gather(b_total):
  b_per_w = b_total // NW
  nchunk = b_per_w // CH
  assert b_per_w % CH == 0 and b_total % (8 * NW) == 0

  mesh = plsc.VectorSubcoreMesh(core_axis_name="c", subcore_axis_name="s")

  @functools.partial(
      pl.kernel,
      mesh=mesh,
      out_type=[
          jax.ShapeDtypeStruct((b_total, EMB), jnp.float32),
          jax.ShapeDtypeStruct((b_total,), jnp.int32),
      ],
      scratch_types=[
          pltpu.VMEM((b_per_w,), jnp.int32),
          pltpu.VMEM((2, CH, EMB), jnp.float32),
          pltpu.VMEM((b_per_w,), jnp.int32),
          pltpu.SemaphoreType.DMA,
          pltpu.SemaphoreType.DMA,
          pltpu.SemaphoreType.DMA,
          pltpu.SemaphoreType.DMA,
      ],
  )
  def gather_kernel(table_hbm, posmap_hbm, ids_hbm, rows_out_hbm, pos_out_hbm,
                    idx_v, rows_v, pos_v, gsem, psem, osem0, osem1):
    wid = lax.axis_index("s") * 2 + lax.axis_index("c")
    base = wid * b_per_w
    pltpu.sync_copy(ids_hbm.at[pl.ds(base, b_per_w)], idx_v)
    osems = (osem0, osem1)
    pending = [None, None]
    for c in range(nchunk):
      s = c % 2
      if pending[s] is not None:
        pending[s].wait()
      idx_c = idx_v.at[pl.ds(c * CH, CH)]
      gp = pltpu.async_copy(posmap_hbm.at[idx_c],
                            pos_v.at[pl.ds(c * CH, CH)], psem)
      g = pltpu.async_copy(table_hbm.at[idx_c], rows_v.at[s], gsem)
      gp.wait()
      g.wait()
      pending[s] = pltpu.async_copy(
          rows_v.at[s], rows_out_hbm.at[pl.ds(base + c * CH, CH)], osems[s])
    for p in pending:
      if p is not None:
        p.wait()
    pltpu.sync_copy(pos_v, pos_out_hbm.at[pl.ds(base, b_per_w)])

  return gather_kernel


def _project_block(x_ref, pidx_ref, poshead_ref, we_ref, wp_ref, b_ref,
                   *rest):
  o_ref = rest[-1]
  x = x_ref[...]
  pidx = pidx_ref[0, 0, :]
  ph = (pidx[:, None] == lax.broadcasted_iota(jnp.int32, (M_BLK, 16), 1)
        ).astype(jnp.float32)
  pp = lax.dot_general(poshead_ref[...], wp_ref[...],
                       (((1,), (1,)), ((), ())),
                       preferred_element_type=jnp.float32)
  acc = lax.dot_general(x, we_ref[...], (((1,), (1,)), ((), ())),
                        preferred_element_type=jnp.float32)
  acc = acc + lax.dot_general(ph, pp, (((1,), (0,)), ((), ())),
                              preferred_element_type=jnp.float32)
  o_ref[...] = acc + b_ref[...]


def _project_chunk(x, pidx3, poshead, we, wp, b2, prev, k, n):
  h = x.shape[0]
  nb = h // M_BLK
  blk0 = k * nb
  pdim = wp.shape[1]
  in_specs = [
      pl.BlockSpec((M_BLK, EMB), lambda i: (i, 0)),
      pl.BlockSpec((1, 1, M_BLK), lambda i: (i, 0, 0)),
      pl.BlockSpec((16, pdim), lambda i: (0, 0)),
      pl.BlockSpec((OUT, EMB), lambda i: (0, 0)),
      pl.BlockSpec((OUT, pdim), lambda i: (0, 0)),
      pl.BlockSpec((1, OUT), lambda i: (0, 0)),
  ]
  args = [x, pidx3, poshead, we, wp, b2]
  aliases = {}
  if prev is not None:
    in_specs.append(pl.BlockSpec(memory_space=pltpu.HBM))
    args.append(prev)
    aliases = {6: 0}
  return pl.pallas_call(
      _project_block,
      grid=(nb,),
      in_specs=in_specs,
      out_specs=pl.BlockSpec((M_BLK, OUT), lambda i: (blk0 + i, 0)),
      out_shape=jax.ShapeDtypeStruct((n, OUT), jnp.float32),
      input_output_aliases=aliases,
      compiler_params=pltpu.CompilerParams(
          dimension_semantics=("arbitrary",)),
  )(*args)


def kernel(entity_ids, entity_table, pos_table, entity_id_to_pos_index, W, b):
  nb_, nc, ne = entity_ids.shape
  n = entity_ids.size
  ids = jnp.transpose(entity_ids, (0, 2, 1)).reshape(-1).astype(jnp.int32)
  posmap = entity_id_to_pos_index.astype(jnp.int32)
  we = W[:, :EMB]
  wp = W[:, EMB:]
  poshead = pos_table[:16]
  b2 = b.reshape(1, OUT)
  h = n // K_SPLIT
  gf = _make_sc_gather(h)
  chunks = [
      gf(entity_table, posmap, lax.slice(ids, (k * h,), ((k + 1) * h,)))
      for k in range(K_SPLIT)
  ]
  out = None
  for k, (rows_k, pidx_k) in enumerate(chunks):
    out = _project_chunk(rows_k, pidx_k.reshape(h // M_BLK, 1, M_BLK),
                         poshead, we, wp, b2, out, k, n)
  return out.reshape(nb_, ne, nc, OUT).transpose(0, 2, 1, 3)
